# baseline (speedup 1.0000x reference)
import jax
import jax.numpy as jnp
from jax.experimental import pallas as pl
from jax.experimental.pallas import tpu as pltpu


def _body(s_r, e_r, dec_r, q_r,
          wave_b, don_b, oh_b, ohp_b, lam_r,
          out_w, out_oh):
    i = pl.program_id(0)
    s = s_r[i]
    e = e_r[i]
    d = dec_r[i]
    L = wave_b.shape[-1]
    pos = jax.lax.broadcasted_iota(jnp.int32, (1, 1, L), 2)
    m = (pos >= s) & (pos < e) & (d == 1)
    out_w[...] = jnp.where(m, don_b[...], wave_b[...])
    lam = lam_r[i]
    mix = lam * oh_b[...] + (1.0 - lam) * ohp_b[...]
    out_oh[...] = jnp.where(d == 1, mix, oh_b[...])


def kernel(wave, onehot_machine, lam, dec, perm, start):
    B, L = wave.shape
    C = onehot_machine.shape[1]

    crop = ((1.0 - lam) * L).astype(jnp.int32)
    max_start = jnp.maximum(1, L - crop)
    s = jnp.mod(start, max_start)
    e = s + crop
    deci = dec.astype(jnp.int32)
    active = deci == 1
    q = jnp.where(active, perm, -1)
    q = jax.lax.associative_scan(lambda a, b: jnp.where(b < 0, a, b), q)
    q = jnp.where(q < 0, 0, q)

    wave3 = wave.reshape(B, 1, L)
    oh3 = onehot_machine.reshape(B, 1, C)

    def self_map(i, s_r, e_r, dec_r, q_r):
        return i, 0, 0

    def donor_map(i, s_r, e_r, dec_r, q_r):
        return q_r[i], 0, 0

    grid_spec = pltpu.PrefetchScalarGridSpec(
        num_scalar_prefetch=4,
        grid=(B,),
        in_specs=[
            pl.BlockSpec((1, 1, L), self_map),
            pl.BlockSpec((1, 1, L), donor_map),
            pl.BlockSpec((1, 1, C), self_map),
            pl.BlockSpec((1, 1, C), donor_map),
            pl.BlockSpec(memory_space=pltpu.SMEM),
        ],
        out_specs=[
            pl.BlockSpec((1, 1, L), self_map),
            pl.BlockSpec((1, 1, C), self_map),
        ],
    )

    out_w, out_oh = pl.pallas_call(
        _body,
        grid_spec=grid_spec,
        out_shape=[
            jax.ShapeDtypeStruct((B, 1, L), jnp.float32),
            jax.ShapeDtypeStruct((B, 1, C), jnp.float32),
        ],
    )(s, e, deci, q, wave3, wave3, oh3, oh3, lam)
    return out_w.reshape(B, L), out_oh.reshape(B, C)


# speedup vs baseline: 1.0968x; 1.0968x over previous
import jax
import jax.numpy as jnp
from jax import lax
from jax.experimental import pallas as pl
from jax.experimental.pallas import tpu as pltpu
from jax.experimental.pallas import tpu_sc as plsc

_NCORE = 2
_CH = 16000
_NCH = 10


def _sc_body(B, L, C):
    rows_per_w = B // 32

    def body(s_hbm, e_hbm, dec_hbm, perm_hbm, lam_hbm, wave_hbm, oh_hbm,
             out_w, out_oh,
             sbuf, ebuf, dbuf, pbuf, lambuf, main, evec, ohs, ohd,
             ldsem, stsem, bsem):
        w = lax.axis_index("s") * _NCORE + lax.axis_index("c")
        pltpu.sync_copy(s_hbm, sbuf)
        pltpu.sync_copy(e_hbm, ebuf)
        pltpu.sync_copy(dec_hbm, dbuf)
        pltpu.sync_copy(perm_hbm, pbuf)
        pltpu.sync_copy(lam_hbm, lambuf)
        lanes = lax.iota(jnp.int32, 16)

        def pick(buf, row):
            base = (row // 16) * 16
            v = buf[pl.ds(pl.multiple_of(base, 16), 16)]
            return jnp.sum(jnp.where(lanes == row - base, v,
                                     jnp.zeros_like(v)))

        def do_label(row):
            p = pick(pbuf, row)
            dec = pick(dbuf, row)
            lamv = pick(lambuf, row)
            f = jnp.where(dec == 1, lamv, 1.0)
            pltpu.sync_copy(oh_hbm.at[row], ohs)
            pltpu.sync_copy(oh_hbm.at[p], ohd)

            def lbody(i, _):
                x = ohs[pl.ds(pl.multiple_of(i * 16, 16), 16)]
                y = ohd[pl.ds(pl.multiple_of(i * 16, 16), 16)]
                ohs[pl.ds(pl.multiple_of(i * 16, 16), 16)] = f * x + (1.0 - f) * y
                return 0

            lax.fori_loop(0, C // 16, lbody, 0)
            pltpu.sync_copy(ohs, out_oh.at[row])

        def do_row(row):
            s = pick(sbuf, row)
            e = pick(ebuf, row)
            dec = pick(dbuf, row)
            p = pick(pbuf, row)
            wl = jnp.where(dec == 1, e - s, 0)
            ee = s + wl
            wl_u = wl.astype(jnp.uint32)

            def load_cp(k):
                lo = k * _CH
                full = (s <= lo) & (ee >= lo + _CH)
                src = jnp.where(full, p, row)
                slot = lax.rem(k, 2)
                return pltpu.make_async_copy(
                    wave_hbm.at[src, pl.ds(pl.multiple_of(lo, 16), _CH)],
                    main.at[pl.ds(pl.multiple_of(slot * _CH, 16), _CH)],
                    ldsem.at[slot])

            def store_cp(k):
                lo = k * _CH
                slot = lax.rem(k, 2)
                return pltpu.make_async_copy(
                    main.at[pl.ds(pl.multiple_of(slot * _CH, 16), _CH)],
                    out_w.at[row, pl.ds(pl.multiple_of(lo, 16), _CH)],
                    stsem.at[slot])

            def boundary(k):
                lo = k * _CH
                slot = lax.rem(k, 2)
                full = (s <= lo) & (ee >= lo + _CH)
                inter = (s < lo + _CH) & (ee > lo)
                partial = inter & jnp.logical_not(full) & (wl > 0)

                @pl.when(partial)
                def _():
                    a = jnp.maximum(s, lo)
                    b = jnp.minimum(ee, lo + _CH)
                    a16 = ((a + 15) // 16) * 16
                    b16 = (b // 16) * 16
                    mid = jnp.maximum(b16 - a16, 0)

                    def mid_cps(do_start):
                        cur = a16
                        for bit in range(13, 3, -1):
                            sz = 1 << bit
                            take = ((mid >> bit) & 1) == 1

                            @pl.when(take)
                            def _(cur=cur, sz=sz):
                                cp = pltpu.make_async_copy(
                                    wave_hbm.at[p, pl.ds(pl.multiple_of(cur, 16), sz)],
                                    main.at[pl.ds(pl.multiple_of(slot * _CH + cur - lo, 16), sz)],
                                    bsem)
                                if do_start:
                                    cp.start()
                                else:
                                    cp.wait()
                            cur = cur + jnp.where(take, sz, 0)

                    mid_cps(True)
                    mid_cps(False)

                    def edge(v, cond):
                        @pl.when(cond)
                        def _():
                            pltpu.sync_copy(
                                wave_hbm.at[p, pl.ds(pl.multiple_of(v * 16, 16), 16)], evec)
                            pos = lanes + v * 16
                            m = (pos - s).astype(jnp.uint32) < wl_u
                            off = slot * _CH + v * 16 - lo
                            curv = main[pl.ds(pl.multiple_of(off, 16), 16)]
                            main[pl.ds(pl.multiple_of(off, 16), 16)] = jnp.where(m, evec[...],
                                                             curv)

                    va = a // 16
                    vb = (b - 1) // 16
                    edge(va, partial)
                    edge(vb, vb != va)

            load_cp(0).start()

            def cb(k, _):
                load_cp(k).wait()
                boundary(k)
                store_cp(k).start()

                @pl.when(k >= 1)
                def _():
                    store_cp(k - 1).wait()

                @pl.when(k + 1 < _NCH)
                def _():
                    load_cp(k + 1).start()

                return 0

            lax.fori_loop(0, _NCH, cb, 0)
            store_cp(_NCH - 1).wait()

        for j in range(rows_per_w):
            row = w * rows_per_w + j
            do_label(row)
            do_row(row)

    return body


def kernel(wave, onehot_machine, lam, dec, perm, start):
    B, L = wave.shape
    C = onehot_machine.shape[1]

    crop = ((1.0 - lam) * L).astype(jnp.int32)
    max_start = jnp.maximum(1, L - crop)
    s = jnp.mod(start, max_start)
    e = s + crop
    deci = dec.astype(jnp.int32)

    mesh = plsc.VectorSubcoreMesh(core_axis_name="c", subcore_axis_name="s")
    run = pl.kernel(
        _sc_body(B, L, C),
        out_type=[
            jax.ShapeDtypeStruct((B, L), jnp.float32),
            jax.ShapeDtypeStruct((B, C), jnp.float32),
        ],
        mesh=mesh,
        compiler_params=pltpu.CompilerParams(use_tc_tiling_on_sc=False, needs_layout_passes=False),
        scratch_types=[
            pltpu.VMEM((B,), jnp.int32),
            pltpu.VMEM((B,), jnp.int32),
            pltpu.VMEM((B,), jnp.int32),
            pltpu.VMEM((B,), jnp.int32),
            pltpu.VMEM((B,), jnp.float32),
            pltpu.VMEM((2 * _CH,), jnp.float32),
            pltpu.VMEM((16,), jnp.float32),
            pltpu.VMEM((C,), jnp.float32),
            pltpu.VMEM((C,), jnp.float32),
            pltpu.SemaphoreType.DMA((2,)),
            pltpu.SemaphoreType.DMA((2,)),
            pltpu.SemaphoreType.DMA,
        ],
    )
    out_w, out_oh = run(s, e, deci, perm, lam, wave, onehot_machine)
    return out_w, out_oh


# speedup vs baseline: 1.1521x; 1.0504x over previous
import jax
import jax.numpy as jnp
from jax import lax
from jax.experimental import pallas as pl
from jax.experimental.pallas import tpu as pltpu
from jax.experimental.pallas import tpu_sc as plsc

_NCORE = 2
_CH = 32000
_NCH = 5


def _sc_body(B, L, C):
    rows_per_w = B // 32

    def body(s_hbm, e_hbm, dec_hbm, perm_hbm, lam_hbm, wave_hbm, oh_hbm,
             out_w, out_oh,
             sbuf, ebuf, dbuf, pbuf, lambuf, main, evec, ohs, ohd,
             ldsem, stsem, bsem):
        w = lax.axis_index("s") * _NCORE + lax.axis_index("c")
        pltpu.sync_copy(s_hbm, sbuf)
        pltpu.sync_copy(e_hbm, ebuf)
        pltpu.sync_copy(dec_hbm, dbuf)
        pltpu.sync_copy(perm_hbm, pbuf)
        pltpu.sync_copy(lam_hbm, lambuf)
        lanes = lax.iota(jnp.int32, 16)

        def pick(buf, row):
            base = (row // 16) * 16
            v = buf[pl.ds(pl.multiple_of(base, 16), 16)]
            return jnp.sum(jnp.where(lanes == row - base, v,
                                     jnp.zeros_like(v)))

        def do_label(row):
            p = pick(pbuf, row)
            dec = pick(dbuf, row)
            lamv = pick(lambuf, row)
            f = jnp.where(dec == 1, lamv, 1.0)
            pltpu.sync_copy(oh_hbm.at[row], ohs)
            pltpu.sync_copy(oh_hbm.at[p], ohd)

            def lbody(i, _):
                x = ohs[pl.ds(pl.multiple_of(i * 16, 16), 16)]
                y = ohd[pl.ds(pl.multiple_of(i * 16, 16), 16)]
                ohs[pl.ds(pl.multiple_of(i * 16, 16), 16)] = f * x + (1.0 - f) * y
                return 0

            lax.fori_loop(0, C // 16, lbody, 0)
            pltpu.sync_copy(ohs, out_oh.at[row])

        def do_row(row):
            s = pick(sbuf, row)
            e = pick(ebuf, row)
            dec = pick(dbuf, row)
            p = pick(pbuf, row)
            wl = jnp.where(dec == 1, e - s, 0)
            ee = s + wl
            wl_u = wl.astype(jnp.uint32)

            def load_cp(k):
                lo = k * _CH
                full = (s <= lo) & (ee >= lo + _CH)
                src = jnp.where(full, p, row)
                slot = lax.rem(k, 2)
                return pltpu.make_async_copy(
                    wave_hbm.at[src, pl.ds(pl.multiple_of(lo, 16), _CH)],
                    main.at[pl.ds(pl.multiple_of(slot * _CH, 16), _CH)],
                    ldsem.at[slot])

            def store_cp(k):
                lo = k * _CH
                slot = lax.rem(k, 2)
                return pltpu.make_async_copy(
                    main.at[pl.ds(pl.multiple_of(slot * _CH, 16), _CH)],
                    out_w.at[row, pl.ds(pl.multiple_of(lo, 16), _CH)],
                    stsem.at[slot])

            def boundary(k):
                lo = k * _CH
                slot = lax.rem(k, 2)
                full = (s <= lo) & (ee >= lo + _CH)
                inter = (s < lo + _CH) & (ee > lo)
                partial = inter & jnp.logical_not(full) & (wl > 0)

                @pl.when(partial)
                def _():
                    a = jnp.maximum(s, lo)
                    b = jnp.minimum(ee, lo + _CH)
                    a16 = ((a + 15) // 16) * 16
                    b16 = (b // 16) * 16
                    mid = jnp.maximum(b16 - a16, 0)

                    def mid_cps(do_start):
                        cur = a16
                        for bit in range((_CH - 1).bit_length() - 1, 3, -1):
                            sz = 1 << bit
                            take = ((mid >> bit) & 1) == 1

                            @pl.when(take)
                            def _(cur=cur, sz=sz):
                                cp = pltpu.make_async_copy(
                                    wave_hbm.at[p, pl.ds(pl.multiple_of(cur, 16), sz)],
                                    main.at[pl.ds(pl.multiple_of(slot * _CH + cur - lo, 16), sz)],
                                    bsem)
                                if do_start:
                                    cp.start()
                                else:
                                    cp.wait()
                            cur = cur + jnp.where(take, sz, 0)

                    mid_cps(True)
                    mid_cps(False)

                    def edge(v, cond):
                        @pl.when(cond)
                        def _():
                            pltpu.sync_copy(
                                wave_hbm.at[p, pl.ds(pl.multiple_of(v * 16, 16), 16)], evec)
                            pos = lanes + v * 16
                            m = (pos - s).astype(jnp.uint32) < wl_u
                            off = slot * _CH + v * 16 - lo
                            curv = main[pl.ds(pl.multiple_of(off, 16), 16)]
                            main[pl.ds(pl.multiple_of(off, 16), 16)] = jnp.where(m, evec[...],
                                                             curv)

                    va = a // 16
                    vb = (b - 1) // 16
                    edge(va, partial)
                    edge(vb, vb != va)

            load_cp(0).start()
            do_label(row)

            def cb(k, _):
                load_cp(k).wait()
                boundary(k)
                store_cp(k).start()

                @pl.when(k >= 1)
                def _():
                    store_cp(k - 1).wait()

                @pl.when(k + 1 < _NCH)
                def _():
                    load_cp(k + 1).start()

                return 0

            lax.fori_loop(0, _NCH, cb, 0)
            store_cp(_NCH - 1).wait()

        for j in range(rows_per_w):
            row = w * rows_per_w + j
            do_row(row)

    return body


def kernel(wave, onehot_machine, lam, dec, perm, start):
    B, L = wave.shape
    C = onehot_machine.shape[1]

    crop = ((1.0 - lam) * L).astype(jnp.int32)
    max_start = jnp.maximum(1, L - crop)
    s = jnp.mod(start, max_start)
    e = s + crop
    deci = dec.astype(jnp.int32)

    mesh = plsc.VectorSubcoreMesh(core_axis_name="c", subcore_axis_name="s")
    run = pl.kernel(
        _sc_body(B, L, C),
        out_type=[
            jax.ShapeDtypeStruct((B, L), jnp.float32),
            jax.ShapeDtypeStruct((B, C), jnp.float32),
        ],
        mesh=mesh,
        compiler_params=pltpu.CompilerParams(use_tc_tiling_on_sc=False, needs_layout_passes=False),
        scratch_types=[
            pltpu.VMEM((B,), jnp.int32),
            pltpu.VMEM((B,), jnp.int32),
            pltpu.VMEM((B,), jnp.int32),
            pltpu.VMEM((B,), jnp.int32),
            pltpu.VMEM((B,), jnp.float32),
            pltpu.VMEM((2 * _CH,), jnp.float32),
            pltpu.VMEM((16,), jnp.float32),
            pltpu.VMEM((C,), jnp.float32),
            pltpu.VMEM((C,), jnp.float32),
            pltpu.SemaphoreType.DMA((2,)),
            pltpu.SemaphoreType.DMA((2,)),
            pltpu.SemaphoreType.DMA,
        ],
    )
    out_w, out_oh = run(s, e, deci, perm, lam, wave, onehot_machine)
    return out_w, out_oh


# speedup vs baseline: 4.5694x; 3.9663x over previous
import jax
import jax.numpy as jnp
from jax.experimental import pallas as pl
from jax.experimental.pallas import tpu as pltpu

_G = 8
_CH = 160000
_PC = 16000


def _make_body(B, L, C, NC):
    TOT = (B // _G) * NC

    def donor_dma(s_r, e_r, dec_r, perm_r, wave_hbm, dbuf, dsem, g, c, slot,
                  do_start):
        for r in range(_G):
            row = g * _G + r
            s = s_r[row]
            e = e_r[row]
            active = (dec_r[row] == 1) & (e > s)
            s128 = (s // 128) * 128
            e128 = ((e + 127) // 128) * 128
            cnt = jnp.where(active, (e128 - s128 + _PC - 1) // _PC, 0)
            p = perm_r[row]

            def piece(j, _, r=r, p=p, s128=s128):
                st = pl.multiple_of(jnp.minimum(s128 + j * _PC, L - _PC), 128)
                cp = pltpu.make_async_copy(
                    wave_hbm.at[p, pl.ds(st, _PC)],
                    dbuf.at[slot, r, pl.ds(st, _PC)],
                    dsem.at[slot, r],
                )
                if do_start:
                    cp.start()
                else:
                    cp.wait()
                return 0

            jax.lax.fori_loop(0, cnt, piece, 0)

    def label_dma(dec_r, perm_r, oh_hbm, ohbuf, ohsem, g, do_start):
        for r in range(_G):
            row = g * _G + r

            @pl.when(dec_r[row] == 1)
            def _():
                cp = pltpu.make_async_copy(
                    oh_hbm.at[perm_r[row]], ohbuf.at[r], ohsem.at[r])
                if do_start:
                    cp.start()
                else:
                    cp.wait()

    def body(s_r, e_r, dec_r, perm_r,
             wave_b, wave_hbm, oh_b, oh_hbm, lam_r,
             out_w, out_oh,
             dbuf, ohbuf, dsem, ohsem):
        g = pl.program_id(0)
        c = pl.program_id(1)
        step = g * NC + c
        slot = jax.lax.rem(step, 2)

        @pl.when(step == 0)
        def _():
            donor_dma(s_r, e_r, dec_r, perm_r, wave_hbm, dbuf, dsem,
                      g, c, slot, True)

        @pl.when(c == 0)
        def _():
            label_dma(dec_r, perm_r, oh_hbm, ohbuf, ohsem, g, True)

        @pl.when(step + 1 < TOT)
        def _():
            wrap = c + 1 == NC
            gn = jnp.where(wrap, g + 1, g)
            cn = jnp.where(wrap, 0, c + 1)
            donor_dma(s_r, e_r, dec_r, perm_r, wave_hbm, dbuf, dsem,
                      gn, cn, 1 - slot, True)

        donor_dma(s_r, e_r, dec_r, perm_r, wave_hbm, dbuf, dsem,
                  g, c, slot, False)

        lo = c * _CH
        svec = jnp.stack([s_r[g * _G + r] for r in range(_G)]).reshape(_G, 1)
        evec = jnp.stack([e_r[g * _G + r] for r in range(_G)]).reshape(_G, 1)
        dvec = jnp.stack([dec_r[g * _G + r] for r in range(_G)]).reshape(_G, 1)
        lenvec = jnp.where(dvec == 1, evec - svec, 0).astype(jnp.uint32)
        any_need = jnp.any((lenvec > 0) & (svec < lo + _CH) & (evec > lo))

        @pl.when(any_need)
        def _():
            pos = jax.lax.broadcasted_iota(jnp.int32, (_G, _CH), 1) + lo
            m = (pos - svec).astype(jnp.uint32) < lenvec
            out_w[...] = jnp.where(m, dbuf[slot], wave_b[...])

        @pl.when(jnp.logical_not(any_need))
        def _():
            out_w[...] = wave_b[...]

        @pl.when(c == NC - 1)
        def _():
            label_dma(dec_r, perm_r, oh_hbm, ohbuf, ohsem, g, False)
            lamv = jnp.stack(
                [lam_r[g * _G + r] for r in range(_G)]).reshape(_G, 1)
            mix = lamv * oh_b[...] + (1.0 - lamv) * ohbuf[...]
            out_oh[...] = jnp.where(dvec == 1, mix, oh_b[...])

    return body


def kernel(wave, onehot_machine, lam, dec, perm, start):
    B, L = wave.shape
    C = onehot_machine.shape[1]
    NC = L // _CH

    crop = ((1.0 - lam) * L).astype(jnp.int32)
    max_start = jnp.maximum(1, L - crop)
    s = jnp.mod(start, max_start)
    e = s + crop
    deci = dec.astype(jnp.int32)

    def wave_map(g, c, *_):
        return g, c

    def oh_map(g, c, *_):
        return g, 0

    grid_spec = pltpu.PrefetchScalarGridSpec(
        num_scalar_prefetch=4,
        grid=(B // _G, NC),
        in_specs=[
            pl.BlockSpec((_G, _CH), wave_map),
            pl.BlockSpec(memory_space=pl.ANY),
            pl.BlockSpec((_G, C), oh_map),
            pl.BlockSpec(memory_space=pl.ANY),
            pl.BlockSpec(memory_space=pltpu.SMEM),
        ],
        out_specs=[
            pl.BlockSpec((_G, _CH), wave_map),
            pl.BlockSpec((_G, C), oh_map),
        ],
        scratch_shapes=[
            pltpu.VMEM((2, _G, _CH), jnp.float32),
            pltpu.VMEM((_G, C), jnp.float32),
            pltpu.SemaphoreType.DMA((2, _G)),
            pltpu.SemaphoreType.DMA((_G,)),
        ],
    )

    out_w, out_oh = pl.pallas_call(
        _make_body(B, L, C, NC),
        grid_spec=grid_spec,
        out_shape=[
            jax.ShapeDtypeStruct((B, L), jnp.float32),
            jax.ShapeDtypeStruct((B, C), jnp.float32),
        ],
    )(s, e, deci, perm, wave, wave, onehot_machine, onehot_machine, lam)
    return out_w, out_oh
